# baseline (speedup 1.0000x reference)
import functools

import jax
import jax.numpy as jnp
from jax import lax
from jax.experimental import pallas as pl
from jax.experimental.pallas import tpu as pltpu
from jax.experimental.pallas import tpu_sc as plsc

_NC = 2
_NS = 16
_NW = _NC * _NS
_LANES = 16


@functools.lru_cache(maxsize=None)
def _make_sc_gather_sum(B, L, V2, D):
    D2 = 2 * D
    ROWS = B // _NW
    NBUF = 4
    NVR = D // _LANES
    LPAD = (L + _LANES - 1) // _LANES * _LANES
    ABLK = 64
    GPB = ABLK // NBUF
    assert B % _NW == 0 and ROWS % ABLK == 0 and ABLK % NBUF == 0
    assert L <= 128 and D % _LANES == 0

    mesh = plsc.VectorSubcoreMesh(core_axis_name="c", subcore_axis_name="s")

    @functools.partial(
        pl.kernel,
        mesh=mesh,
        out_type=jax.ShapeDtypeStruct((B, D), jnp.float32),
        scratch_types=[
            pltpu.VMEM((ROWS, LPAD), jnp.int32),
            pltpu.VMEM((NBUF, LPAD), jnp.int32),
            pltpu.VMEM((NBUF, L, D2), jnp.float32),
            pltpu.VMEM((ABLK, D), jnp.float32),
        ] + [pltpu.SemaphoreType.DMA] * NBUF,
    )
    def sc_gather_sum(x64_hbm, tab2_hbm, out_hbm,
                      idx_v, ring_v, rows_v, acc_v, *sems):
        wid = lax.axis_index("s") * _NC + lax.axis_index("c")
        base = wid * ROWS
        pltpu.sync_copy(x64_hbm.at[pl.ds(base, ROWS)], idx_v)

        def start(j, b):
            for grp in range(LPAD // _LANES):
                lo = grp * _LANES
                ring_v[b, pl.ds(lo, _LANES)] = (
                    idx_v[j, pl.ds(lo, _LANES)] >> 1)
            pltpu.async_copy(
                tab2_hbm.at[ring_v.at[b, pl.ds(0, L)]], rows_v.at[b], sems[b])

        for b in range(NBUF):
            start(b, b)

        def outer(g, carry):
            for b in range(NBUF):
                j = g * NBUF + b
                pltpu.make_async_copy(
                    tab2_hbm.at[ring_v.at[b, pl.ds(0, L)]],
                    rows_v.at[b], sems[b]).wait()
                jn = j + NBUF

                @pl.when(jn < ROWS)
                def _():
                    start(jn, b)

                accs = [jnp.zeros((_LANES,), jnp.float32)] * NVR
                for grp in range(LPAD // _LANES):
                    lo = grp * _LANES
                    offv = (idx_v[j, pl.ds(lo, _LANES)] & 1) * D
                    for u in range(min(_LANES, L - lo)):
                        off = offv[u]
                        for q in range(NVR):
                            accs[q] = accs[q] + rows_v[
                                b, lo + u, pl.ds(off + q * _LANES, _LANES)]
                arow = (g % GPB) * NBUF + b
                for q in range(NVR):
                    acc_v[arow, pl.ds(q * _LANES, _LANES)] = accs[q]

            @pl.when(g % GPB == GPB - 1)
            def _():
                blk = g // GPB
                pltpu.sync_copy(
                    acc_v, out_hbm.at[pl.ds(base + blk * ABLK, ABLK)])

            return carry

        lax.fori_loop(0, ROWS // NBUF, outer, 0)

    return sc_gather_sum


@functools.lru_cache(maxsize=None)
def _make_tc_mlp(B, L, D, H, C):
    BLK = 512
    assert B % BLK == 0

    def body(x_ref, s_ref, w1_ref, b1_ref, w2_ref, b2_ref, o_ref):
        cnt = jnp.sum((x_ref[...] != 0).astype(jnp.float32), axis=1,
                      keepdims=True)
        avg = s_ref[...] / jnp.maximum(cnt, 1e-6)
        h = jnp.dot(avg, w1_ref[...], preferred_element_type=jnp.float32)
        h = jnp.maximum(h + b1_ref[...], 0.0)
        o_ref[...] = (jnp.dot(h, w2_ref[...],
                              preferred_element_type=jnp.float32)
                      + b2_ref[...])

    return pl.pallas_call(
        body,
        grid=(B // BLK,),
        in_specs=[
            pl.BlockSpec((BLK, L), lambda i: (i, 0)),
            pl.BlockSpec((BLK, D), lambda i: (i, 0)),
            pl.BlockSpec((D, H), lambda i: (0, 0)),
            pl.BlockSpec((1, H), lambda i: (0, 0)),
            pl.BlockSpec((H, C), lambda i: (0, 0)),
            pl.BlockSpec((1, C), lambda i: (0, 0)),
        ],
        out_specs=pl.BlockSpec((BLK, C), lambda i: (i, 0)),
        out_shape=jax.ShapeDtypeStruct((B, C), jnp.float32),
    )


def kernel(x, table, W1, b1, W2, b2):
    B, L = x.shape
    V, D = table.shape
    H = W1.shape[1]
    C = W2.shape[1]
    xi = x.astype(jnp.int32)
    lpad = (L + 15) // 16 * 16
    x64 = jnp.pad(xi, ((0, 0), (0, lpad - L)))
    tab2 = table.reshape(V // 2, 2 * D)
    summed = _make_sc_gather_sum(B, L, V // 2, D)(x64, tab2)
    out = _make_tc_mlp(B, L, D, H, C)(
        xi, summed, W1, b1.reshape(1, H), W2, b2.reshape(1, C))
    return out
